# baseline (speedup 1.0000x reference)
import functools

import jax
import jax.numpy as jnp
from jax import lax
from jax.experimental import pallas as pl
from jax.experimental.pallas import tpu as pltpu
from jax.experimental.pallas import tpu_sc as plsc

N = 1000
B = 16384
NC = 1
NS = 16
LANES = 16
NW = NC * NS
CHUNK = B // NW
NPAD = 1024


def _tc_lse_body(w_ref, wa_ref, t_ref):
    w = w_ref[...]
    m = jnp.max(w, axis=1, keepdims=True)
    s = jnp.sum(jnp.exp(w - m), axis=1, keepdims=True)
    lse_rows = m + jnp.log(s)
    wa = wa_ref[...]
    ma = jnp.max(wa)
    sa = jnp.sum(jnp.exp(wa - ma))
    lse_a = ma + jnp.log(sa)
    comb = wa - lse_a - lse_rows
    t = jnp.pad(w + comb, ((0, 0), (0, NPAD - N)))
    t_ref[...] = t.reshape(N, 8, 128)


def _sc_gather(ab_hbm, t_hbm, out_hbm, ab_v, flat_v, g_v, sem_g):
    wid = lax.axis_index("s") * NC + lax.axis_index("c")
    pltpu.sync_copy(ab_hbm.at[wid], ab_v)

    copies = []
    for c in range(CHUNK // 128):
        for j in range(c * (128 // LANES), (c + 1) * (128 // LANES)):
            a16 = ab_v[0, pl.ds(j * LANES, LANES)]
            b16 = ab_v[1, pl.ds(j * LANES, LANES)]
            flat_v[pl.ds(j * LANES, LANES)] = a16 * NPAD + b16
        copies.append(pltpu.async_copy(
            t_hbm.at[flat_v.at[pl.ds(c * 128, 128)]],
            g_v.at[pl.ds(c * 128, 128)], sem_g))
    for cp in copies:
        cp.wait()

    pltpu.sync_copy(g_v, out_hbm.at[pl.ds(wid * CHUNK, CHUNK)])


@functools.partial(
    pl.kernel,
    mesh=plsc.VectorSubcoreMesh(core_axis_name="c", subcore_axis_name="s",
                                num_cores=NC),
    out_type=jax.ShapeDtypeStruct((B,), jnp.float32),
    scratch_types=[
        pltpu.VMEM((2, CHUNK), jnp.int32),
        pltpu.VMEM((CHUNK,), jnp.int32),
        pltpu.VMEM((CHUNK,), jnp.float32),
        pltpu.SemaphoreType.DMA,
    ],
)
def _sc_kernel(ab_hbm, t_hbm, out_hbm, ab_v, flat_v, g_v, sem_g):
    _sc_gather(ab_hbm, t_hbm, out_hbm, ab_v, flat_v, g_v, sem_g)


def kernel(inputs, w_A, w_B_A):
    ab = jnp.stack([inputs[:, 0].astype(jnp.int32).reshape(NW, CHUNK),
                    inputs[:, 1].astype(jnp.int32).reshape(NW, CHUNK)],
                   axis=1)

    t = pl.pallas_call(
        _tc_lse_body,
        out_shape=jax.ShapeDtypeStruct((N, 8, NPAD // 8), jnp.float32),
    )(w_B_A, w_A.reshape(N, 1))

    return _sc_kernel(ab, t.reshape(-1))


# speedup vs baseline: 1.0158x; 1.0158x over previous
import functools

import jax
import jax.numpy as jnp
from jax import lax
from jax.experimental import pallas as pl
from jax.experimental.pallas import tpu as pltpu
from jax.experimental.pallas import tpu_sc as plsc

N = 1000
B = 16384
NC = 1
NS = 16
LANES = 16
NW = NC * NS
CHUNK = B // NW
NPAD = 1024


def _tc_lse_body(w_ref, wa_ref, t_ref):
    w = w_ref[...]
    m = jnp.max(w, axis=1, keepdims=True)
    s = jnp.sum(jnp.exp(w - m), axis=1, keepdims=True)
    lse_rows = m + jnp.log(s)
    wa = wa_ref[...]
    ma = jnp.max(wa)
    sa = jnp.sum(jnp.exp(wa - ma))
    lse_a = ma + jnp.log(sa)
    comb = wa - lse_a - lse_rows
    t = jnp.pad(w + comb, ((0, 0), (0, NPAD - N)))
    t_ref[...] = t.reshape(N, 8, 128)


def _sc_gather(ab_hbm, t_hbm, out_hbm, ab_v, flat_v, g_v, sem_g, sem_o):
    wid = lax.axis_index("s") * NC + lax.axis_index("c")
    pltpu.sync_copy(ab_hbm.at[wid], ab_v)

    copies = []
    for c in range(CHUNK // 128):
        for j in range(c * (128 // LANES), (c + 1) * (128 // LANES)):
            a16 = ab_v[0, pl.ds(j * LANES, LANES)]
            b16 = ab_v[1, pl.ds(j * LANES, LANES)]
            flat_v[pl.ds(j * LANES, LANES)] = a16 * NPAD + b16
        copies.append(pltpu.async_copy(
            t_hbm.at[flat_v.at[pl.ds(c * 128, 128)]],
            g_v.at[pl.ds(c * 128, 128)], sem_g.at[c]))

    outs = []
    for c in range(CHUNK // 128):
        copies[c].wait()
        outs.append(pltpu.async_copy(
            g_v.at[pl.ds(c * 128, 128)],
            out_hbm.at[pl.ds(wid * CHUNK + c * 128, 128)], sem_o.at[c]))
    for cp in outs:
        cp.wait()


@functools.partial(
    pl.kernel,
    mesh=plsc.VectorSubcoreMesh(core_axis_name="c", subcore_axis_name="s",
                                num_cores=NC),
    out_type=jax.ShapeDtypeStruct((B,), jnp.float32),
    scratch_types=[
        pltpu.VMEM((2, CHUNK), jnp.int32),
        pltpu.VMEM((CHUNK,), jnp.int32),
        pltpu.VMEM((CHUNK,), jnp.float32),
        pltpu.SemaphoreType.DMA((CHUNK // 128,)),
        pltpu.SemaphoreType.DMA((CHUNK // 128,)),
    ],
)
def _sc_kernel(ab_hbm, t_hbm, out_hbm, ab_v, flat_v, g_v, sem_g, sem_o):
    _sc_gather(ab_hbm, t_hbm, out_hbm, ab_v, flat_v, g_v, sem_g, sem_o)


def kernel(inputs, w_A, w_B_A):
    ab = jnp.stack([inputs[:, 0].astype(jnp.int32).reshape(NW, CHUNK),
                    inputs[:, 1].astype(jnp.int32).reshape(NW, CHUNK)],
                   axis=1)

    t = pl.pallas_call(
        _tc_lse_body,
        out_shape=jax.ShapeDtypeStruct((N, 8, NPAD // 8), jnp.float32),
    )(w_B_A, w_A.reshape(N, 1))

    return _sc_kernel(ab, t.reshape(-1))
